# baseline (speedup 1.0000x reference)
import functools

import jax
import jax.numpy as jnp
from jax import lax
from jax.experimental import pallas as pl
from jax.experimental.pallas import tpu as pltpu
from jax.experimental.pallas import tpu_sc as plsc

B = 2
C = 4
N = 128 * 128 * 128
K = max(1, int(N * 0.1))
EPS = 1e-05

NW = 32
NV = N // NW
CHUNK = 2048
NBLK = NV // CHUNK
VREGS = CHUNK // 16
NB = 256
HCOLS = C * 4 * NB


def _sc_body(pred_hbm, targ_hbm, zeros_hbm, hist_hbm,
             pbuf0, pbuf1, tbuf0, tbuf1, hist_v, sem0, sem1):
    wid = lax.axis_index("s") * 2 + lax.axis_index("c")
    base = wid * NV
    iota = lax.iota(jnp.int32, 16)
    ones = jnp.ones((16,), jnp.float32)
    zvec = jnp.zeros((16,), jnp.float32)

    def start_blk(b, blk, pbuf, tbuf, sem):
        off = base + blk * CHUNK
        for cc in range(C):
            pltpu.async_copy(pred_hbm.at[b * C + cc, pl.ds(off, CHUNK)],
                             pbuf.at[cc], sem)
            pltpu.async_copy(targ_hbm.at[b * C + cc, pl.ds(off, CHUNK)],
                             tbuf.at[cc], sem)

    def wait_blk(b, blk, pbuf, tbuf, sem):
        off = base + blk * CHUNK
        for cc in range(C):
            pltpu.make_async_copy(pred_hbm.at[b * C + cc, pl.ds(off, CHUNK)],
                                  pbuf.at[cc], sem).wait()
            pltpu.make_async_copy(targ_hbm.at[b * C + cc, pl.ds(off, CHUNK)],
                                  tbuf.at[cc], sem).wait()

    def compute(pbuf, tbuf):
        @plsc.parallel_loop(0, VREGS, unroll=3)
        def vbody(v):
            sl = pl.ds(v * 16, 16)
            e0 = jnp.exp(pbuf[0, sl])
            e1 = jnp.exp(pbuf[1, sl])
            e2 = jnp.exp(pbuf[2, sl])
            e3 = jnp.exp(pbuf[3, sl])
            inv = 1.0 / (e0 + e1 + e2 + e3)
            es = (e0, e1, e2, e3)
            for cc in range(C):
                p = es[cc] * inv
                t = tbuf[cc, sl]
                err = jnp.abs(p - t)
                f = jnp.minimum(err * float(NB), float(NB - 1))
                col = f.astype(jnp.int32) + (cc * 4 * NB)
                p2 = p * p
                p2t = p2 * t
                plsc.addupdate_scatter(hist_v, [col, iota], ones)
                plsc.addupdate_scatter(hist_v, [col + NB, iota], p2t)
                plsc.addupdate_scatter(hist_v, [col + 2 * NB, iota], p2)
                plsc.addupdate_scatter(hist_v, [col + 3 * NB, iota], t)

    for b in range(B):
        pltpu.sync_copy(zeros_hbm, hist_v)
        start_blk(b, 0, pbuf0, tbuf0, sem0)

        def pair(pp, carry, b=b):
            blk0 = pp * 2
            start_blk(b, blk0 + 1, pbuf1, tbuf1, sem1)
            wait_blk(b, blk0, pbuf0, tbuf0, sem0)
            compute(pbuf0, tbuf0)

            @pl.when(pp < NBLK // 2 - 1)
            def _prefetch():
                start_blk(b, blk0 + 2, pbuf0, tbuf0, sem0)

            wait_blk(b, blk0 + 1, pbuf1, tbuf1, sem1)
            compute(pbuf1, tbuf1)
            return carry

        lax.fori_loop(0, NBLK // 2, pair, 0)
        pltpu.sync_copy(hist_v, hist_hbm.at[wid, b])


def _sc_pass(pred_r, targ_r, zeros):
    mesh = plsc.VectorSubcoreMesh(core_axis_name="c", subcore_axis_name="s")
    return pl.kernel(
        _sc_body,
        out_type=jax.ShapeDtypeStruct((NW, B, HCOLS, 16), jnp.float32),
        mesh=mesh,
        compiler_params=pltpu.CompilerParams(use_tc_tiling_on_sc=False,
                                             needs_layout_passes=False),
        scratch_types=[
            pltpu.VMEM((C, CHUNK), jnp.float32),
            pltpu.VMEM((C, CHUNK), jnp.float32),
            pltpu.VMEM((C, CHUNK), jnp.float32),
            pltpu.VMEM((C, CHUNK), jnp.float32),
            pltpu.VMEM((HCOLS, 16), jnp.float32),
            pltpu.SemaphoreType.DMA,
            pltpu.SemaphoreType.DMA,
        ],
    )(pred_r, targ_r, zeros)


def _epilogue_body(hist_ref, out_ref, acc_ref):
    i = pl.program_id(0)
    h = hist_ref[0]

    @pl.when(i == 0)
    def _():
        acc_ref[...] = h

    @pl.when(i > 0)
    def _():
        acc_ref[...] = acc_ref[...] + h

    @pl.when(i == NW - 1)
    def _():
        acc = jnp.sum(acc_ref[...], axis=-1)
        r_i = lax.broadcasted_iota(jnp.int32, (NB, NB), 0)
        c_i = lax.broadcasted_iota(jnp.int32, (NB, NB), 1)
        U = (r_i >= c_i).astype(jnp.float32)
        jidx = lax.broadcasted_iota(jnp.int32, (B, NB), 1)
        kf = jnp.float32(K)
        num = jnp.zeros((B, 1), jnp.float32)
        nv = jnp.zeros((B, 1), jnp.float32)
        for cc in range(C):
            seg = acc[:, cc * 4 * NB:(cc + 1) * 4 * NB]
            cnt = seg[:, 0:NB]
            p2t = seg[:, NB:2 * NB]
            p2 = seg[:, 2 * NB:3 * NB]
            st = seg[:, 3 * NB:4 * NB]
            A = jnp.dot(cnt, U, preferred_element_type=jnp.float32,
                        precision=lax.Precision.HIGHEST)
            mask = (A >= kf).astype(jnp.float32)
            nbt = jnp.sum(mask.astype(jnp.int32), axis=1, keepdims=True)
            onehot = (jidx == nbt - 1).astype(jnp.float32)
            cnt_b = jnp.sum(cnt * onehot, axis=1, keepdims=True)
            A_b = jnp.sum(A * onehot, axis=1, keepdims=True)
            r = kf - (A_b - cnt_b)
            frac = r / cnt_b
            w = (1.0 - mask) + frac * onehot
            inter = jnp.sum(p2t * w, axis=1, keepdims=True)
            union = (jnp.sum(p2 * w, axis=1, keepdims=True)
                     + jnp.sum(st * w, axis=1, keepdims=True))
            dice = (2.0 * inter + EPS) / (union + EPS)
            loss_c = 1.0 - dice
            tsum_c = jnp.sum(st, axis=1, keepdims=True)
            valid = (tsum_c > 0).astype(jnp.float32)
            num = num + loss_c * valid
            nv = nv + valid
        per_batch = num / jnp.maximum(nv, 1.0)
        bv = (nv > 0).astype(jnp.float32)
        nbv = jnp.sum(bv, axis=0, keepdims=True)
        tot = jnp.sum(per_batch * bv, axis=0, keepdims=True)
        out_ref[...] = tot / jnp.maximum(nbv, 1.0)


def _epilogue(hist):
    return pl.pallas_call(
        _epilogue_body,
        grid=(NW,),
        in_specs=[
            pl.BlockSpec((1, B, HCOLS, 16), lambda i: (i, 0, 0, 0)),
        ],
        out_specs=pl.BlockSpec((1, 1), lambda i: (0, 0)),
        out_shape=jax.ShapeDtypeStruct((1, 1), jnp.float32),
        scratch_shapes=[
            pltpu.VMEM((B, HCOLS, 16), jnp.float32),
        ],
    )(hist)


def kernel(pred, target):
    pred_r = pred.reshape(B * C, N)
    targ_r = target.reshape(B * C, N)
    zeros = jnp.zeros((HCOLS, 16), jnp.float32)
    hist = _sc_pass(pred_r, targ_r, zeros)
    loss = _epilogue(hist)
    return loss[0, 0]


# speedup vs baseline: 1.0777x; 1.0777x over previous
import functools

import jax
import jax.numpy as jnp
from jax import lax
from jax.experimental import pallas as pl
from jax.experimental.pallas import tpu as pltpu
from jax.experimental.pallas import tpu_sc as plsc

B = 2
C = 4
N = 128 * 128 * 128
K = max(1, int(N * 0.1))
EPS = 1e-05

NW = 32
NV = N // NW
CHUNK = 2048
NBLK = NV // CHUNK
VREGS = CHUNK // 16
NB = 256
HCOLS = C * 4 * NB


def _sc_body(pred_hbm, targ_hbm, zeros_hbm, hist_hbm,
             pbuf0, pbuf1, tbuf0, tbuf1, hist_v, sem0, sem1):
    wid = lax.axis_index("s") * 2 + lax.axis_index("c")
    base = wid * NV
    iota = lax.iota(jnp.int32, 16)
    ones = jnp.ones((16,), jnp.float32)
    zvec = jnp.zeros((16,), jnp.float32)

    def start_blk(b, blk, pbuf, tbuf, sem):
        off = base + blk * CHUNK
        for cc in range(C):
            pltpu.async_copy(pred_hbm.at[b * C + cc, pl.ds(off, CHUNK)],
                             pbuf.at[cc], sem)
            pltpu.async_copy(targ_hbm.at[b * C + cc, pl.ds(off, CHUNK)],
                             tbuf.at[cc], sem)

    def wait_blk(b, blk, pbuf, tbuf, sem):
        off = base + blk * CHUNK
        for cc in range(C):
            pltpu.make_async_copy(pred_hbm.at[b * C + cc, pl.ds(off, CHUNK)],
                                  pbuf.at[cc], sem).wait()
            pltpu.make_async_copy(targ_hbm.at[b * C + cc, pl.ds(off, CHUNK)],
                                  tbuf.at[cc], sem).wait()

    def compute(pbuf, tbuf):
        @plsc.parallel_loop(0, VREGS, unroll=1)
        def vbody(v):
            sl = pl.ds(v * 16, 16)
            e0 = jnp.exp(pbuf[0, sl])
            e1 = jnp.exp(pbuf[1, sl])
            e2 = jnp.exp(pbuf[2, sl])
            e3 = jnp.exp(pbuf[3, sl])
            inv = 1.0 / (e0 + e1 + e2 + e3)
            es = (e0, e1, e2, e3)
            for cc in range(C):
                p = es[cc] * inv
                t = tbuf[cc, sl]
                err = jnp.abs(p - t)
                f = jnp.minimum(err * float(NB), float(NB - 1))
                col = f.astype(jnp.int32) + (cc * 4 * NB)
                p2 = p * p
                p2t = p2 * t
                plsc.addupdate_scatter(hist_v, [col, iota], ones)
                plsc.addupdate_scatter(hist_v, [col + NB, iota], p2t)
                plsc.addupdate_scatter(hist_v, [col + 2 * NB, iota], p2)
                plsc.addupdate_scatter(hist_v, [col + 3 * NB, iota], t)

    for b in range(B):
        pltpu.sync_copy(zeros_hbm, hist_v)
        start_blk(b, 0, pbuf0, tbuf0, sem0)

        def pair(pp, carry, b=b):
            blk0 = pp * 2
            start_blk(b, blk0 + 1, pbuf1, tbuf1, sem1)
            wait_blk(b, blk0, pbuf0, tbuf0, sem0)
            compute(pbuf0, tbuf0)

            @pl.when(pp < NBLK // 2 - 1)
            def _prefetch():
                start_blk(b, blk0 + 2, pbuf0, tbuf0, sem0)

            wait_blk(b, blk0 + 1, pbuf1, tbuf1, sem1)
            compute(pbuf1, tbuf1)
            return carry

        lax.fori_loop(0, NBLK // 2, pair, 0)
        pltpu.sync_copy(hist_v, hist_hbm.at[wid, b])


def _sc_pass(pred_r, targ_r, zeros):
    mesh = plsc.VectorSubcoreMesh(core_axis_name="c", subcore_axis_name="s")
    return pl.kernel(
        _sc_body,
        out_type=jax.ShapeDtypeStruct((NW, B, HCOLS, 16), jnp.float32),
        mesh=mesh,
        compiler_params=pltpu.CompilerParams(use_tc_tiling_on_sc=False,
                                             needs_layout_passes=False),
        scratch_types=[
            pltpu.VMEM((C, CHUNK), jnp.float32),
            pltpu.VMEM((C, CHUNK), jnp.float32),
            pltpu.VMEM((C, CHUNK), jnp.float32),
            pltpu.VMEM((C, CHUNK), jnp.float32),
            pltpu.VMEM((HCOLS, 16), jnp.float32),
            pltpu.SemaphoreType.DMA,
            pltpu.SemaphoreType.DMA,
        ],
    )(pred_r, targ_r, zeros)


def _epilogue_body(hist_ref, out_ref, acc_ref):
    i = pl.program_id(0)
    h = hist_ref[0]

    @pl.when(i == 0)
    def _():
        acc_ref[...] = h

    @pl.when(i > 0)
    def _():
        acc_ref[...] = acc_ref[...] + h

    @pl.when(i == NW - 1)
    def _():
        acc = jnp.sum(acc_ref[...], axis=-1)
        r_i = lax.broadcasted_iota(jnp.int32, (NB, NB), 0)
        c_i = lax.broadcasted_iota(jnp.int32, (NB, NB), 1)
        U = (r_i >= c_i).astype(jnp.float32)
        jidx = lax.broadcasted_iota(jnp.int32, (B, NB), 1)
        kf = jnp.float32(K)
        num = jnp.zeros((B, 1), jnp.float32)
        nv = jnp.zeros((B, 1), jnp.float32)
        for cc in range(C):
            seg = acc[:, cc * 4 * NB:(cc + 1) * 4 * NB]
            cnt = seg[:, 0:NB]
            p2t = seg[:, NB:2 * NB]
            p2 = seg[:, 2 * NB:3 * NB]
            st = seg[:, 3 * NB:4 * NB]
            A = jnp.dot(cnt, U, preferred_element_type=jnp.float32,
                        precision=lax.Precision.HIGHEST)
            mask = (A >= kf).astype(jnp.float32)
            nbt = jnp.sum(mask.astype(jnp.int32), axis=1, keepdims=True)
            onehot = (jidx == nbt - 1).astype(jnp.float32)
            cnt_b = jnp.sum(cnt * onehot, axis=1, keepdims=True)
            A_b = jnp.sum(A * onehot, axis=1, keepdims=True)
            r = kf - (A_b - cnt_b)
            frac = r / cnt_b
            w = (1.0 - mask) + frac * onehot
            inter = jnp.sum(p2t * w, axis=1, keepdims=True)
            union = (jnp.sum(p2 * w, axis=1, keepdims=True)
                     + jnp.sum(st * w, axis=1, keepdims=True))
            dice = (2.0 * inter + EPS) / (union + EPS)
            loss_c = 1.0 - dice
            tsum_c = jnp.sum(st, axis=1, keepdims=True)
            valid = (tsum_c > 0).astype(jnp.float32)
            num = num + loss_c * valid
            nv = nv + valid
        per_batch = num / jnp.maximum(nv, 1.0)
        bv = (nv > 0).astype(jnp.float32)
        nbv = jnp.sum(bv, axis=0, keepdims=True)
        tot = jnp.sum(per_batch * bv, axis=0, keepdims=True)
        out_ref[...] = tot / jnp.maximum(nbv, 1.0)


def _epilogue(hist):
    return pl.pallas_call(
        _epilogue_body,
        grid=(NW,),
        in_specs=[
            pl.BlockSpec((1, B, HCOLS, 16), lambda i: (i, 0, 0, 0)),
        ],
        out_specs=pl.BlockSpec((1, 1), lambda i: (0, 0)),
        out_shape=jax.ShapeDtypeStruct((1, 1), jnp.float32),
        scratch_shapes=[
            pltpu.VMEM((B, HCOLS, 16), jnp.float32),
        ],
    )(hist)


def kernel(pred, target):
    pred_r = pred.reshape(B * C, N)
    targ_r = target.reshape(B * C, N)
    zeros = jnp.zeros((HCOLS, 16), jnp.float32)
    hist = _sc_pass(pred_r, targ_r, zeros)
    loss = _epilogue(hist)
    return loss[0, 0]


# speedup vs baseline: 1.1376x; 1.0555x over previous
import functools

import jax
import jax.numpy as jnp
from jax import lax
from jax.experimental import pallas as pl
from jax.experimental.pallas import tpu as pltpu
from jax.experimental.pallas import tpu_sc as plsc

B = 2
C = 4
N = 128 * 128 * 128
K = max(1, int(N * 0.1))
EPS = 1e-05

NW = 32
NV = N // NW
CHUNK = 2048
NBLK = NV // CHUNK
VREGS = CHUNK // 16
NB = 256
MAX_LT_ONE = float.fromhex("0x1.fffffep-1")
HCOLS = C * 4 * NB


def _sc_body(pred_hbm, targ_hbm, zeros_hbm, hist_hbm,
             pbuf0, pbuf1, tbuf0, tbuf1, hist_v, sem0, sem1):
    wid = lax.axis_index("s") * 2 + lax.axis_index("c")
    base = wid * NV
    iota = lax.iota(jnp.int32, 16)
    ones = jnp.ones((16,), jnp.float32)
    zvec = jnp.zeros((16,), jnp.float32)

    def start_blk(b, blk, pbuf, tbuf, sem):
        off = base + blk * CHUNK
        for cc in range(C):
            pltpu.async_copy(pred_hbm.at[b * C + cc, pl.ds(off, CHUNK)],
                             pbuf.at[cc], sem)
            pltpu.async_copy(targ_hbm.at[b * C + cc, pl.ds(off, CHUNK)],
                             tbuf.at[cc], sem)

    def wait_blk(b, blk, pbuf, tbuf, sem):
        off = base + blk * CHUNK
        for cc in range(C):
            pltpu.make_async_copy(pred_hbm.at[b * C + cc, pl.ds(off, CHUNK)],
                                  pbuf.at[cc], sem).wait()
            pltpu.make_async_copy(targ_hbm.at[b * C + cc, pl.ds(off, CHUNK)],
                                  tbuf.at[cc], sem).wait()

    def compute(pbuf, tbuf):
        @plsc.parallel_loop(0, VREGS, unroll=2)
        def vbody(v):
            sl = pl.ds(v * 16, 16)
            e0 = jnp.exp(pbuf[0, sl])
            e1 = jnp.exp(pbuf[1, sl])
            e2 = jnp.exp(pbuf[2, sl])
            e3 = jnp.exp(pbuf[3, sl])
            inv = 1.0 / (e0 + e1 + e2 + e3)
            es = (e0, e1, e2, e3)
            for cc in range(C):
                p = es[cc] * inv
                t = tbuf[cc, sl]
                err = jnp.abs(p - t)
                err = jnp.minimum(err, MAX_LT_ONE)
                u = plsc.bitcast(err + 1.0, jnp.int32)
                col = (u >> 15) + (cc * 4 * NB - 0x7F00)
                p2 = p * p
                p2t = p2 * t
                plsc.addupdate_scatter(hist_v, [col, iota], ones)
                plsc.addupdate_scatter(hist_v, [col + NB, iota], p2t)
                plsc.addupdate_scatter(hist_v, [col + 2 * NB, iota], p2)
                plsc.addupdate_scatter(hist_v, [col + 3 * NB, iota], t)

    for b in range(B):
        pltpu.sync_copy(zeros_hbm, hist_v)
        start_blk(b, 0, pbuf0, tbuf0, sem0)

        def pair(pp, carry, b=b):
            blk0 = pp * 2
            start_blk(b, blk0 + 1, pbuf1, tbuf1, sem1)
            wait_blk(b, blk0, pbuf0, tbuf0, sem0)
            compute(pbuf0, tbuf0)

            @pl.when(pp < NBLK // 2 - 1)
            def _prefetch():
                start_blk(b, blk0 + 2, pbuf0, tbuf0, sem0)

            wait_blk(b, blk0 + 1, pbuf1, tbuf1, sem1)
            compute(pbuf1, tbuf1)
            return carry

        lax.fori_loop(0, NBLK // 2, pair, 0)
        pltpu.sync_copy(hist_v, hist_hbm.at[wid, b])


def _sc_pass(pred_r, targ_r, zeros):
    mesh = plsc.VectorSubcoreMesh(core_axis_name="c", subcore_axis_name="s")
    return pl.kernel(
        _sc_body,
        out_type=jax.ShapeDtypeStruct((NW, B, HCOLS, 16), jnp.float32),
        mesh=mesh,
        compiler_params=pltpu.CompilerParams(use_tc_tiling_on_sc=False,
                                             needs_layout_passes=False),
        scratch_types=[
            pltpu.VMEM((C, CHUNK), jnp.float32),
            pltpu.VMEM((C, CHUNK), jnp.float32),
            pltpu.VMEM((C, CHUNK), jnp.float32),
            pltpu.VMEM((C, CHUNK), jnp.float32),
            pltpu.VMEM((HCOLS, 16), jnp.float32),
            pltpu.SemaphoreType.DMA,
            pltpu.SemaphoreType.DMA,
        ],
    )(pred_r, targ_r, zeros)


def _epilogue_body(hist_ref, out_ref, acc_ref):
    i = pl.program_id(0)
    h = hist_ref[0]

    @pl.when(i == 0)
    def _():
        acc_ref[...] = h

    @pl.when(i > 0)
    def _():
        acc_ref[...] = acc_ref[...] + h

    @pl.when(i == NW - 1)
    def _():
        acc = jnp.sum(acc_ref[...], axis=-1)
        r_i = lax.broadcasted_iota(jnp.int32, (NB, NB), 0)
        c_i = lax.broadcasted_iota(jnp.int32, (NB, NB), 1)
        U = (r_i >= c_i).astype(jnp.float32)
        jidx = lax.broadcasted_iota(jnp.int32, (B, NB), 1)
        kf = jnp.float32(K)
        num = jnp.zeros((B, 1), jnp.float32)
        nv = jnp.zeros((B, 1), jnp.float32)
        for cc in range(C):
            seg = acc[:, cc * 4 * NB:(cc + 1) * 4 * NB]
            cnt = seg[:, 0:NB]
            p2t = seg[:, NB:2 * NB]
            p2 = seg[:, 2 * NB:3 * NB]
            st = seg[:, 3 * NB:4 * NB]
            A = jnp.dot(cnt, U, preferred_element_type=jnp.float32,
                        precision=lax.Precision.HIGHEST)
            mask = (A >= kf).astype(jnp.float32)
            nbt = jnp.sum(mask.astype(jnp.int32), axis=1, keepdims=True)
            onehot = (jidx == nbt - 1).astype(jnp.float32)
            cnt_b = jnp.sum(cnt * onehot, axis=1, keepdims=True)
            A_b = jnp.sum(A * onehot, axis=1, keepdims=True)
            r = kf - (A_b - cnt_b)
            frac = r / cnt_b
            w = (1.0 - mask) + frac * onehot
            inter = jnp.sum(p2t * w, axis=1, keepdims=True)
            union = (jnp.sum(p2 * w, axis=1, keepdims=True)
                     + jnp.sum(st * w, axis=1, keepdims=True))
            dice = (2.0 * inter + EPS) / (union + EPS)
            loss_c = 1.0 - dice
            tsum_c = jnp.sum(st, axis=1, keepdims=True)
            valid = (tsum_c > 0).astype(jnp.float32)
            num = num + loss_c * valid
            nv = nv + valid
        per_batch = num / jnp.maximum(nv, 1.0)
        bv = (nv > 0).astype(jnp.float32)
        nbv = jnp.sum(bv, axis=0, keepdims=True)
        tot = jnp.sum(per_batch * bv, axis=0, keepdims=True)
        out_ref[...] = tot / jnp.maximum(nbv, 1.0)


def _epilogue(hist):
    return pl.pallas_call(
        _epilogue_body,
        grid=(NW,),
        in_specs=[
            pl.BlockSpec((1, B, HCOLS, 16), lambda i: (i, 0, 0, 0)),
        ],
        out_specs=pl.BlockSpec((1, 1), lambda i: (0, 0)),
        out_shape=jax.ShapeDtypeStruct((1, 1), jnp.float32),
        scratch_shapes=[
            pltpu.VMEM((B, HCOLS, 16), jnp.float32),
        ],
    )(hist)


def kernel(pred, target):
    pred_r = pred.reshape(B * C, N)
    targ_r = target.reshape(B * C, N)
    zeros = jnp.zeros((HCOLS, 16), jnp.float32)
    hist = _sc_pass(pred_r, targ_r, zeros)
    loss = _epilogue(hist)
    return loss[0, 0]


# speedup vs baseline: 1.4477x; 1.2726x over previous
import functools

import jax
import jax.numpy as jnp
from jax import lax
from jax.experimental import pallas as pl
from jax.experimental.pallas import tpu as pltpu
from jax.experimental.pallas import tpu_sc as plsc

B = 2
C = 4
N = 128 * 128 * 128
K = max(1, int(N * 0.1))
EPS = 1e-05

NW = 32
NV = N // NW
CHUNK = 2048
NBLK = NV // CHUNK
VREGS = CHUNK // 16
NB = 256
MAX_LT_ONE = float.fromhex("0x1.fffffep-1")
HCOLS = C * 4 * NB


def _sc_body(pred_hbm, targ_hbm, zeros_hbm, hist_hbm,
             pbuf0, pbuf1, tbuf0, tbuf1, hist_v, sem0, sem1):
    wid = lax.axis_index("s") * 2 + lax.axis_index("c")
    base = wid * NV
    iota = lax.iota(jnp.int32, 16)
    ones = jnp.ones((16,), jnp.float32)
    zvec = jnp.zeros((16,), jnp.float32)

    def start_blk(b, blk, pbuf, tbuf, sem):
        off = base + blk * CHUNK
        for cc in range(C):
            pltpu.async_copy(pred_hbm.at[b * C + cc, pl.ds(off, CHUNK)],
                             pbuf.at[cc], sem)
            pltpu.async_copy(targ_hbm.at[b * C + cc, pl.ds(off, CHUNK)],
                             tbuf.at[cc], sem)

    def wait_blk(b, blk, pbuf, tbuf, sem):
        off = base + blk * CHUNK
        for cc in range(C):
            pltpu.make_async_copy(pred_hbm.at[b * C + cc, pl.ds(off, CHUNK)],
                                  pbuf.at[cc], sem).wait()
            pltpu.make_async_copy(targ_hbm.at[b * C + cc, pl.ds(off, CHUNK)],
                                  tbuf.at[cc], sem).wait()

    def compute(pbuf, tbuf):
        @plsc.parallel_loop(0, VREGS, unroll=2)
        def vbody(v):
            sl = pl.ds(v * 16, 16)
            e0 = jnp.exp(pbuf[0, sl])
            e1 = jnp.exp(pbuf[1, sl])
            e2 = jnp.exp(pbuf[2, sl])
            e3 = jnp.exp(pbuf[3, sl])
            inv = 1.0 / (e0 + e1 + e2 + e3)
            es = (e0, e1, e2, e3)
            for cc in range(C):
                p = es[cc] * inv
                t = tbuf[cc, sl]
                err = jnp.abs(p - t)
                err = jnp.minimum(err, MAX_LT_ONE)
                u = plsc.bitcast(err + 1.0, jnp.int32)
                col = (u >> 15) + (cc * 4 * NB - 0x7F00)
                idx = (col << 4) + iota
                p2 = p * p
                p2t = p2 * t
                plsc.addupdate_scatter(hist_v, [idx], ones)
                plsc.addupdate_scatter(hist_v, [idx + NB * 16], p2t)
                plsc.addupdate_scatter(hist_v, [idx + 2 * NB * 16], p2)
                plsc.addupdate_scatter(hist_v, [idx + 3 * NB * 16], t)

    for b in range(B):
        pltpu.sync_copy(zeros_hbm, hist_v)
        start_blk(b, 0, pbuf0, tbuf0, sem0)

        def pair(pp, carry, b=b):
            blk0 = pp * 2
            start_blk(b, blk0 + 1, pbuf1, tbuf1, sem1)
            wait_blk(b, blk0, pbuf0, tbuf0, sem0)
            compute(pbuf0, tbuf0)

            @pl.when(pp < NBLK // 2 - 1)
            def _prefetch():
                start_blk(b, blk0 + 2, pbuf0, tbuf0, sem0)

            wait_blk(b, blk0 + 1, pbuf1, tbuf1, sem1)
            compute(pbuf1, tbuf1)
            return carry

        lax.fori_loop(0, NBLK // 2, pair, 0)
        pltpu.sync_copy(hist_v,
                        hist_hbm.at[pl.ds((wid * B + b) * HCOLS * 16,
                                          HCOLS * 16)])


def _sc_pass(pred_r, targ_r, zeros):
    mesh = plsc.VectorSubcoreMesh(core_axis_name="c", subcore_axis_name="s")
    return pl.kernel(
        _sc_body,
        out_type=jax.ShapeDtypeStruct((NW * B * HCOLS * 16,), jnp.float32),
        mesh=mesh,
        compiler_params=pltpu.CompilerParams(use_tc_tiling_on_sc=False,
                                             needs_layout_passes=False),
        scratch_types=[
            pltpu.VMEM((C, CHUNK), jnp.float32),
            pltpu.VMEM((C, CHUNK), jnp.float32),
            pltpu.VMEM((C, CHUNK), jnp.float32),
            pltpu.VMEM((C, CHUNK), jnp.float32),
            pltpu.VMEM((HCOLS * 16,), jnp.float32),
            pltpu.SemaphoreType.DMA,
            pltpu.SemaphoreType.DMA,
        ],
    )(pred_r, targ_r, zeros)


def _epilogue_body(hist_ref, out_ref, acc_ref):
    g = pl.program_id(0)
    h = hist_ref[0]

    for b in range(B):
        @pl.when(jnp.logical_and(g % B == b, g < B))
        def _():
            acc_ref[b] = h

        @pl.when(jnp.logical_and(g % B == b, g >= B))
        def _():
            acc_ref[b] = acc_ref[b] + h

    @pl.when(g == NW * B - 1)
    def _():
        k_i = lax.broadcasted_iota(jnp.int32, (128, 8), 0)
        j_i = lax.broadcasted_iota(jnp.int32, (128, 8), 1)
        W = ((k_i >> 4) == j_i).astype(jnp.float32)
        c_a = lax.broadcasted_iota(jnp.int32, (8, 8), 0)
        c_b = lax.broadcasted_iota(jnp.int32, (8, 8), 1)
        U8 = (c_a >= c_b).astype(jnp.float32)
        r_a = lax.broadcasted_iota(jnp.int32, (32, 32), 0)
        r_b = lax.broadcasted_iota(jnp.int32, (32, 32), 1)
        M32 = (r_b > r_a).astype(jnp.float32)
        ones8 = jnp.ones((8, 1), jnp.float32)
        iota_flat = (lax.broadcasted_iota(jnp.int32, (32, 8), 0) * 8
                     + lax.broadcasted_iota(jnp.int32, (32, 8), 1))
        kf = jnp.float32(K)
        hp = lax.Precision.HIGHEST

        num = []
        nv = []
        for b in range(B):
            folded = jnp.dot(acc_ref[b], W, precision=hp,
                             preferred_element_type=jnp.float32)
            num_b = jnp.float32(0.0)
            nv_b = jnp.float32(0.0)
            for cc in range(C):
                base = cc * 128
                cnt = folded[base:base + 32]
                p2t = folded[base + 32:base + 64]
                p2 = folded[base + 64:base + 96]
                st = folded[base + 96:base + 128]
                rowsum = jnp.dot(cnt, ones8, precision=hp,
                                 preferred_element_type=jnp.float32)
                tail = jnp.dot(M32, rowsum, precision=hp,
                               preferred_element_type=jnp.float32)
                A = jnp.dot(cnt, U8, precision=hp,
                            preferred_element_type=jnp.float32) + tail
                mask = (A >= kf).astype(jnp.float32)
                nb = jnp.sum(mask.astype(jnp.int32))
                onehot = (iota_flat == nb - 1).astype(jnp.float32)
                cnt_b = jnp.sum(cnt * onehot)
                A_b = jnp.sum(A * onehot)
                r = kf - (A_b - cnt_b)
                frac = r / cnt_b
                w = (1.0 - mask) + frac * onehot
                inter = jnp.sum(p2t * w)
                union = jnp.sum(p2 * w) + jnp.sum(st * w)
                dice = (2.0 * inter + EPS) / (union + EPS)
                loss_c = 1.0 - dice
                valid = (jnp.sum(st) > 0).astype(jnp.float32)
                num_b = num_b + loss_c * valid
                nv_b = nv_b + valid
            num.append(num_b)
            nv.append(nv_b)
        tot = jnp.float32(0.0)
        nbv = jnp.float32(0.0)
        for b in range(B):
            per_batch = num[b] / jnp.maximum(nv[b], 1.0)
            bv = (nv[b] > 0).astype(jnp.float32)
            tot = tot + per_batch * bv
            nbv = nbv + bv
        loss = tot / jnp.maximum(nbv, 1.0)
        out_ref[...] = lax.broadcast_in_dim(loss, (1, 1), ())


def _epilogue(hist):
    return pl.pallas_call(
        _epilogue_body,
        grid=(NW * B,),
        in_specs=[
            pl.BlockSpec((1, HCOLS * 16 // 128, 128), lambda i: (i, 0, 0)),
        ],
        out_specs=pl.BlockSpec((1, 1), lambda i: (0, 0)),
        out_shape=jax.ShapeDtypeStruct((1, 1), jnp.float32),
        scratch_shapes=[
            pltpu.VMEM((B, HCOLS * 16 // 128, 128), jnp.float32),
        ],
    )(hist)


def kernel(pred, target):
    pred_r = pred.reshape(B * C, N)
    targ_r = target.reshape(B * C, N)
    zeros = jnp.zeros((HCOLS * 16,), jnp.float32)
    hist = _sc_pass(pred_r, targ_r, zeros)
    loss = _epilogue(hist.reshape(NW * B, HCOLS * 16 // 128, 128))
    return loss[0, 0]
